# baseline (speedup 1.0000x reference)
import jax
import jax.numpy as jnp
from jax import lax
from jax.experimental import pallas as pl
from jax.experimental.pallas import tpu as pltpu

_CB = 24


def _rev8_within(data, axis):
    h = data.shape[axis]
    phase = lax.broadcasted_iota(jnp.int32, data.shape, axis)
    for s in (4, 2, 1):
        up = pltpu.roll(data, h - s, axis)
        dn = pltpu.roll(data, s, axis)
        data = jnp.where((phase & s) == 0, up, dn)
    return data


def _flip_block(data):
    r8 = _rev8_within(data, 1)
    nt = data.shape[1] // 8
    return jnp.concatenate(
        [r8[:, 8 * (nt - 1 - j):8 * (nt - 1 - j) + 8] for j in range(nt)], axis=1
    )


def _shuffle(c, ncb):
    return c


def _flip_body(mask_ref, x_ref, maskv_ref, o_ref):
    ncb = pl.num_programs(1)
    c = _shuffle(pl.program_id(1), ncb)
    base = c * _CB
    count = mask_ref[base]
    for ch in range(1, _CB):
        count += mask_ref[base + ch]

    @pl.when(count == 0)
    def _copy():
        o_ref[0] = x_ref[0]

    @pl.when(count == _CB)
    def _flip_all():
        data = x_ref[0]
        r8 = _rev8_within(data, 1)
        nt = data.shape[1] // 8
        for j in range(nt):
            src = 8 * (nt - 1 - j)
            o_ref[0, :, pl.ds(8 * j, 8)] = r8[:, src:src + 8]

    @pl.when(jnp.logical_and(count > 0, count < _CB))
    def _mixed():
        data = x_ref[0]
        flipped = _flip_block(data)
        mv = maskv_ref[...][:, :, None]
        o_ref[0] = jnp.where(mv != 0, flipped, data)


def _kernel_tc(x, params, indices):
    B, C, H, W = x.shape
    mask = jnp.zeros((C,), jnp.int32).at[indices].set(1)
    mask = mask * params[0].astype(jnp.int32)
    maskv = mask.reshape(C, 1)
    ncb = C // _CB
    grid_spec = pltpu.PrefetchScalarGridSpec(
        num_scalar_prefetch=1,
        grid=(B, ncb),
        in_specs=[
            pl.BlockSpec(
                (1, _CB, H, W),
                lambda b, c, mask_ref: (b, _shuffle(c, ncb), 0, 0),
            ),
            pl.BlockSpec((_CB, 1), lambda b, c, mask_ref: (_shuffle(c, ncb), 0)),
        ],
        out_specs=pl.BlockSpec(
            (1, _CB, H, W),
            lambda b, c, mask_ref: (b, _shuffle(c, ncb), 0, 0),
        ),
    )
    return pl.pallas_call(
        _flip_body,
        grid_spec=grid_spec,
        out_shape=jax.ShapeDtypeStruct(x.shape, x.dtype),
        compiler_params=pltpu.CompilerParams(
            dimension_semantics=("parallel", "parallel"),
        ),
    )(mask, x, maskv)


import functools
from jax.experimental.pallas import tpu_sc as plsc

_T = 128


def _sc_permute_rows(x_rows, idx):
    R, D = x_rows.shape
    info = plsc.get_sparse_core_info()
    NC, NS = info.num_cores, info.num_subcores
    NW = NC * NS
    per_w = R // NW
    nt = per_w // _T
    assert per_w % _T == 0 and nt % 2 == 0
    mesh = plsc.VectorSubcoreMesh(core_axis_name="c", subcore_axis_name="s")

    @functools.partial(
        pl.kernel,
        mesh=mesh,
        out_type=jax.ShapeDtypeStruct((R, D), jnp.float32),
        compiler_params=pltpu.CompilerParams(use_tc_tiling_on_sc=False),
        scratch_types=[
            pltpu.VMEM((per_w,), jnp.int32),
            pltpu.VMEM((_T, D), jnp.float32),
            pltpu.VMEM((_T, D), jnp.float32),
            pltpu.SemaphoreType.DMA,
            pltpu.SemaphoreType.DMA,
        ],
    )
    def k(x_hbm, idx_hbm, out_hbm, idx_v, buf0, buf1, sem0, sem1):
        wid = lax.axis_index("s") * NC + lax.axis_index("c")
        base = wid * per_w
        pltpu.sync_copy(idx_hbm.at[pl.ds(base, per_w)], idx_v)
        bufs = (buf0, buf1)
        sems = (sem0, sem1)

        def gather(tile, buf, sem):
            pltpu.make_async_copy(
                x_hbm.at[idx_v.at[pl.ds(tile * _T, _T)]], buf, sem
            ).start()

        gather(0, buf0, sem0)
        gather(1, buf1, sem1)

        def body(i, carry):
            for p in range(2):
                g = 2 * i + p
                buf, sem = bufs[p], sems[p]
                pltpu.make_async_copy(
                    x_hbm.at[idx_v.at[pl.ds(g * _T, _T)]], buf, sem
                ).wait()
                pltpu.sync_copy(buf, out_hbm.at[pl.ds(base + g * _T, _T)])

                @pl.when(g + 2 < nt)
                def _next():
                    gather(g + 2, buf, sem)

            return carry

        lax.fori_loop(0, nt // 2, body, 0)

    return k(x_rows, idx)


def _kernel_sc(x, params, indices):
    B, C, H, W = x.shape
    mask = jnp.zeros((C,), jnp.bool_).at[indices].set(True)
    mask = jnp.logical_and(mask, params[0])
    R = B * C * H
    r = jnp.arange(R, dtype=jnp.int32)
    h = r % H
    c = (r // H) % C
    src = r - h + jnp.where(mask[c], H - 1 - h, h)
    out_rows = _sc_permute_rows(x.reshape(R, W), src)
    return out_rows.reshape(B, C, H, W)


def kernel(x, params, indices):
    return _kernel_sc(x, params, indices)


# speedup vs baseline: 9.6631x; 9.6631x over previous
import jax
import jax.numpy as jnp
from jax import lax
from jax.experimental import pallas as pl
from jax.experimental.pallas import tpu as pltpu

_CB = 24


def _rev8_within(data, axis):
    h = data.shape[axis]
    phase = lax.broadcasted_iota(jnp.int32, data.shape, axis)
    for s in (4, 2, 1):
        up = pltpu.roll(data, h - s, axis)
        dn = pltpu.roll(data, s, axis)
        data = jnp.where((phase & s) == 0, up, dn)
    return data


def _flip_block(data):
    r8 = _rev8_within(data, 1)
    nt = data.shape[1] // 8
    return jnp.concatenate(
        [r8[:, 8 * (nt - 1 - j):8 * (nt - 1 - j) + 8] for j in range(nt)], axis=1
    )


def _shuffle(c, ncb):
    return c


def _flip_body(mask_ref, x_ref, maskv_ref, o_ref):
    ncb = pl.num_programs(1)
    c = _shuffle(pl.program_id(1), ncb)
    base = c * _CB
    count = mask_ref[base]
    for ch in range(1, _CB):
        count += mask_ref[base + ch]

    @pl.when(count == 0)
    def _copy():
        o_ref[0] = x_ref[0]

    @pl.when(count == _CB)
    def _flip_all():
        data = x_ref[0]
        r8 = _rev8_within(data, 1)
        nt = data.shape[1] // 8
        for j in range(nt):
            src = 8 * (nt - 1 - j)
            o_ref[0, :, pl.ds(8 * j, 8)] = r8[:, src:src + 8]

    @pl.when(jnp.logical_and(count > 0, count < _CB))
    def _mixed():
        data = x_ref[0]
        flipped = _flip_block(data)
        mv = maskv_ref[...][:, :, None]
        o_ref[0] = jnp.where(mv != 0, flipped, data)


def _kernel_tc(x, params, indices):
    B, C, H, W = x.shape
    mask = jnp.zeros((C,), jnp.int32).at[indices].set(1)
    mask = mask * params[0].astype(jnp.int32)
    maskv = mask.reshape(C, 1)
    ncb = C // _CB
    grid_spec = pltpu.PrefetchScalarGridSpec(
        num_scalar_prefetch=1,
        grid=(B, ncb),
        in_specs=[
            pl.BlockSpec(
                (1, _CB, H, W),
                lambda b, c, mask_ref: (b, _shuffle(c, ncb), 0, 0),
            ),
            pl.BlockSpec((_CB, 1), lambda b, c, mask_ref: (_shuffle(c, ncb), 0)),
        ],
        out_specs=pl.BlockSpec(
            (1, _CB, H, W),
            lambda b, c, mask_ref: (b, _shuffle(c, ncb), 0, 0),
        ),
    )
    return pl.pallas_call(
        _flip_body,
        grid_spec=grid_spec,
        out_shape=jax.ShapeDtypeStruct(x.shape, x.dtype),
        compiler_params=pltpu.CompilerParams(
            dimension_semantics=("parallel", "parallel"),
        ),
    )(mask, x, maskv)


import functools
from jax.experimental.pallas import tpu_sc as plsc

_T = 128


def _sc_flip_images(x_imgs, flags):
    G, H, W = x_imgs.shape
    HH = H // 2
    info = plsc.get_sparse_core_info()
    NC, NS = info.num_cores, info.num_subcores
    NW = NC * NS
    per_w = G // NW
    n_units = 2 * per_w
    mesh = plsc.VectorSubcoreMesh(core_axis_name="c", subcore_axis_name="s")

    @functools.partial(
        pl.kernel,
        mesh=mesh,
        out_type=jax.ShapeDtypeStruct((G, H, W), jnp.float32),
        scratch_types=[
            pltpu.VMEM((per_w + 16,), jnp.int32),
            pltpu.VMEM((HH, W), jnp.float32),
            pltpu.VMEM((HH, W), jnp.float32),
            pltpu.SemaphoreType.DMA,
            pltpu.SemaphoreType.DMA,
            pltpu.SemaphoreType.DMA,
            pltpu.SemaphoreType.DMA,
        ],
    )
    def k(x_hbm, flags_hbm, out_hbm, flags_v, buf0, buf1, isem0, isem1,
          osem0, osem1):
        wid = lax.axis_index("s") * NC + lax.axis_index("c")
        gbase = wid * per_w
        pltpu.sync_copy(flags_hbm.at[pl.ds(gbase, per_w)], flags_v.at[pl.ds(0, per_w)])
        bufs = (buf0, buf1)
        isems = (isem0, isem1)
        osems = (osem0, osem1)

        def unit(u):
            return gbase + u // 2, u % 2

        def start_in(u, p):
            g, h = unit(u)
            pltpu.make_async_copy(
                x_hbm.at[g, pl.ds(HH * h, HH)], bufs[p], isems[p]
            ).start()

        def reverse(buf):
            def swap(j, carry):
                for kk in range(W // 16):
                    sl = pl.ds(16 * kk, 16)
                    a = buf[j, sl]
                    b = buf[HH - 1 - j, sl]
                    buf[j, sl] = b
                    buf[HH - 1 - j, sl] = a
                return carry
            lax.fori_loop(0, HH // 2, swap, 0)

        start_in(0, 0)
        start_in(1, 1)

        def body(i, carry):
            for p in range(2):
                u = 2 * i + p
                g, h = unit(u)
                flag = flags_v[pl.ds(u // 2, 16)][0]
                pltpu.make_async_copy(
                    x_hbm.at[g, pl.ds(HH * h, HH)], bufs[p], isems[p]
                ).wait()

                @pl.when(flag != 0)
                def _rev():
                    reverse(bufs[p])

                off = jnp.where(flag != 0, HH * (1 - (u % 2)), HH * (u % 2))
                out_descr = pltpu.make_async_copy(
                    bufs[p], out_hbm.at[g, pl.ds(off, HH)], osems[p]
                )
                out_descr.start()
                out_descr.wait()

                @pl.when(u + 2 < n_units)
                def _nxt():
                    start_in(u + 2, p)

            return carry

        lax.fori_loop(0, n_units // 2, body, 0)

    return k(x_imgs, flags)


def _kernel_sc(x, params, indices):
    B, C, H, W = x.shape
    mask = jnp.zeros((C,), jnp.bool_).at[indices].set(True)
    mask = jnp.logical_and(mask, params[0])
    flags = jnp.tile(mask.astype(jnp.int32), B)
    out = _sc_flip_images(x.reshape(B * C, H, W), flags)
    return out.reshape(B, C, H, W)


def kernel(x, params, indices):
    return _kernel_sc(x, params, indices)


# speedup vs baseline: 11.5670x; 1.1970x over previous
import jax
import jax.numpy as jnp
from jax import lax
from jax.experimental import pallas as pl
from jax.experimental.pallas import tpu as pltpu

_CB = 24


def _rev8_within(data, axis):
    h = data.shape[axis]
    phase = lax.broadcasted_iota(jnp.int32, data.shape, axis)
    for s in (4, 2, 1):
        up = pltpu.roll(data, h - s, axis)
        dn = pltpu.roll(data, s, axis)
        data = jnp.where((phase & s) == 0, up, dn)
    return data


def _flip_block(data):
    r8 = _rev8_within(data, 1)
    nt = data.shape[1] // 8
    return jnp.concatenate(
        [r8[:, 8 * (nt - 1 - j):8 * (nt - 1 - j) + 8] for j in range(nt)], axis=1
    )


def _shuffle(c, ncb):
    return c


def _flip_body(mask_ref, x_ref, maskv_ref, o_ref):
    ncb = pl.num_programs(1)
    c = _shuffle(pl.program_id(1), ncb)
    base = c * _CB
    count = mask_ref[base]
    for ch in range(1, _CB):
        count += mask_ref[base + ch]

    @pl.when(count == 0)
    def _copy():
        o_ref[0] = x_ref[0]

    @pl.when(count == _CB)
    def _flip_all():
        data = x_ref[0]
        r8 = _rev8_within(data, 1)
        nt = data.shape[1] // 8
        for j in range(nt):
            src = 8 * (nt - 1 - j)
            o_ref[0, :, pl.ds(8 * j, 8)] = r8[:, src:src + 8]

    @pl.when(jnp.logical_and(count > 0, count < _CB))
    def _mixed():
        data = x_ref[0]
        flipped = _flip_block(data)
        mv = maskv_ref[...][:, :, None]
        o_ref[0] = jnp.where(mv != 0, flipped, data)


def _kernel_tc(x, params, indices):
    B, C, H, W = x.shape
    mask = jnp.zeros((C,), jnp.int32).at[indices].set(1)
    mask = mask * params[0].astype(jnp.int32)
    maskv = mask.reshape(C, 1)
    ncb = C // _CB
    grid_spec = pltpu.PrefetchScalarGridSpec(
        num_scalar_prefetch=1,
        grid=(B, ncb),
        in_specs=[
            pl.BlockSpec(
                (1, _CB, H, W),
                lambda b, c, mask_ref: (b, _shuffle(c, ncb), 0, 0),
            ),
            pl.BlockSpec((_CB, 1), lambda b, c, mask_ref: (_shuffle(c, ncb), 0)),
        ],
        out_specs=pl.BlockSpec(
            (1, _CB, H, W),
            lambda b, c, mask_ref: (b, _shuffle(c, ncb), 0, 0),
        ),
    )
    return pl.pallas_call(
        _flip_body,
        grid_spec=grid_spec,
        out_shape=jax.ShapeDtypeStruct(x.shape, x.dtype),
        compiler_params=pltpu.CompilerParams(
            dimension_semantics=("parallel", "parallel"),
        ),
    )(mask, x, maskv)


import functools
from jax.experimental.pallas import tpu_sc as plsc

_T = 128


def _sc_flip_images(x_imgs, flags):
    G, H, W = x_imgs.shape
    Q = H // 4
    info = plsc.get_sparse_core_info()
    NC, NS = info.num_cores, info.num_subcores
    NW = NC * NS
    per_w = G // NW
    n_units = 4 * per_w
    mesh = plsc.VectorSubcoreMesh(core_axis_name="c", subcore_axis_name="s")

    @functools.partial(
        pl.kernel,
        mesh=mesh,
        out_type=jax.ShapeDtypeStruct((G, H, W), jnp.float32),
        scratch_types=[
            pltpu.VMEM((per_w + 16,), jnp.int32),
            pltpu.VMEM((Q, W), jnp.float32),
            pltpu.VMEM((Q, W), jnp.float32),
            pltpu.VMEM((Q, W), jnp.float32),
            pltpu.VMEM((Q, W), jnp.float32),
            pltpu.SemaphoreType.DMA,
            pltpu.SemaphoreType.DMA,
            pltpu.SemaphoreType.DMA,
            pltpu.SemaphoreType.DMA,
            pltpu.SemaphoreType.DMA,
            pltpu.SemaphoreType.DMA,
            pltpu.SemaphoreType.DMA,
            pltpu.SemaphoreType.DMA,
        ],
    )
    def k(x_hbm, flags_hbm, out_hbm, flags_v, buf0, buf1, buf2, buf3,
          is0, is1, is2, is3, os0, os1, os2, os3):
        wid = lax.axis_index("s") * NC + lax.axis_index("c")
        gbase = wid * per_w
        pltpu.sync_copy(flags_hbm.at[pl.ds(gbase, per_w)],
                        flags_v.at[pl.ds(0, per_w)])
        bufs = (buf0, buf1, buf2, buf3)
        isems = (is0, is1, is2, is3)
        osems = (os0, os1, os2, os3)

        def in_descr(u, p):
            g = gbase + u // 4
            q = u % 4
            return pltpu.make_async_copy(
                x_hbm.at[g, pl.ds(Q * q, Q)], bufs[p], isems[p]
            )

        def out_descr(u, p, flag):
            g = gbase + u // 4
            q = u % 4
            off = jnp.where(flag != 0, Q * (3 - q), Q * q)
            return pltpu.make_async_copy(
                bufs[p], out_hbm.at[g, pl.ds(off, Q)], osems[p]
            )

        def reverse(buf):
            def swap(j, carry):
                for kk in range(W // 16):
                    sl = pl.ds(16 * kk, 16)
                    a = buf[j, sl]
                    b = buf[Q - 1 - j, sl]
                    buf[j, sl] = b
                    buf[Q - 1 - j, sl] = a
                return carry
            lax.fori_loop(0, Q // 2, swap, 0)

        in_descr(0, 0).start()
        in_descr(1, 1).start()

        def body(i, carry):
            for p in range(4):
                u = 4 * i + p
                flag = flags_v[pl.ds(u // 4, 16)][0]
                in_descr(u, p).wait()

                @pl.when(flag != 0)
                def _rev():
                    reverse(bufs[p])

                out_descr(u, p, flag).start()

                @pl.when(u + 2 < n_units)
                def _prefetch():
                    pn = (p + 2) % 4

                    @pl.when(u >= 2)
                    def _drain():
                        out_descr(u - 2, pn, flags_v[pl.ds((u - 2) // 4, 16)][0]).wait()

                    in_descr(u + 2, pn).start()

            return carry

        lax.fori_loop(0, n_units // 4, body, 0)
        out_descr(n_units - 2, (n_units - 2) % 4,
                  flags_v[pl.ds((n_units - 2) // 4, 16)][0]).wait()
        out_descr(n_units - 1, (n_units - 1) % 4,
                  flags_v[pl.ds((n_units - 1) // 4, 16)][0]).wait()

    return k(x_imgs, flags)


def _kernel_sc(x, params, indices):
    B, C, H, W = x.shape
    mask = jnp.zeros((C,), jnp.bool_).at[indices].set(True)
    mask = jnp.logical_and(mask, params[0])
    flags = jnp.tile(mask.astype(jnp.int32), B)
    out = _sc_flip_images(x.reshape(B * C, H, W), flags)
    return out.reshape(B, C, H, W)


def kernel(x, params, indices):
    return _kernel_sc(x, params, indices)


# speedup vs baseline: 12.9009x; 1.1153x over previous
import jax
import jax.numpy as jnp
from jax import lax
from jax.experimental import pallas as pl
from jax.experimental.pallas import tpu as pltpu

_CB = 24


def _rev8_within(data, axis):
    h = data.shape[axis]
    phase = lax.broadcasted_iota(jnp.int32, data.shape, axis)
    for s in (4, 2, 1):
        up = pltpu.roll(data, h - s, axis)
        dn = pltpu.roll(data, s, axis)
        data = jnp.where((phase & s) == 0, up, dn)
    return data


def _flip_block(data):
    r8 = _rev8_within(data, 1)
    nt = data.shape[1] // 8
    return jnp.concatenate(
        [r8[:, 8 * (nt - 1 - j):8 * (nt - 1 - j) + 8] for j in range(nt)], axis=1
    )


def _shuffle(c, ncb):
    return c


def _flip_body(mask_ref, x_ref, maskv_ref, o_ref):
    ncb = pl.num_programs(1)
    c = _shuffle(pl.program_id(1), ncb)
    base = c * _CB
    count = mask_ref[base]
    for ch in range(1, _CB):
        count += mask_ref[base + ch]

    @pl.when(count == 0)
    def _copy():
        o_ref[0] = x_ref[0]

    @pl.when(count == _CB)
    def _flip_all():
        data = x_ref[0]
        r8 = _rev8_within(data, 1)
        nt = data.shape[1] // 8
        for j in range(nt):
            src = 8 * (nt - 1 - j)
            o_ref[0, :, pl.ds(8 * j, 8)] = r8[:, src:src + 8]

    @pl.when(jnp.logical_and(count > 0, count < _CB))
    def _mixed():
        data = x_ref[0]
        flipped = _flip_block(data)
        mv = maskv_ref[...][:, :, None]
        o_ref[0] = jnp.where(mv != 0, flipped, data)


def _kernel_tc(x, params, indices):
    B, C, H, W = x.shape
    mask = jnp.zeros((C,), jnp.int32).at[indices].set(1)
    mask = mask * params[0].astype(jnp.int32)
    maskv = mask.reshape(C, 1)
    ncb = C // _CB
    grid_spec = pltpu.PrefetchScalarGridSpec(
        num_scalar_prefetch=1,
        grid=(B, ncb),
        in_specs=[
            pl.BlockSpec(
                (1, _CB, H, W),
                lambda b, c, mask_ref: (b, _shuffle(c, ncb), 0, 0),
            ),
            pl.BlockSpec((_CB, 1), lambda b, c, mask_ref: (_shuffle(c, ncb), 0)),
        ],
        out_specs=pl.BlockSpec(
            (1, _CB, H, W),
            lambda b, c, mask_ref: (b, _shuffle(c, ncb), 0, 0),
        ),
    )
    return pl.pallas_call(
        _flip_body,
        grid_spec=grid_spec,
        out_shape=jax.ShapeDtypeStruct(x.shape, x.dtype),
        compiler_params=pltpu.CompilerParams(
            dimension_semantics=("parallel", "parallel"),
        ),
    )(mask, x, maskv)


import functools
from jax.experimental.pallas import tpu_sc as plsc

_T = 128


def _sc_flip_images(x_imgs, flags):
    G, H, W = x_imgs.shape
    D = 8
    NS_IMG = 4
    Q = H // NS_IMG
    info = plsc.get_sparse_core_info()
    NC, NS = info.num_cores, info.num_subcores
    NW = NC * NS
    per_w = G // NW
    n_units = NS_IMG * per_w
    mesh = plsc.VectorSubcoreMesh(core_axis_name="c", subcore_axis_name="s")

    @functools.partial(
        pl.kernel,
        mesh=mesh,
        out_type=jax.ShapeDtypeStruct((G, H, W), jnp.float32),
        scratch_types=[
            pltpu.VMEM((per_w + 16,), jnp.int32),
        ] + [pltpu.VMEM((Q, W), jnp.float32)] * D
          + [pltpu.SemaphoreType.DMA] * (2 * D),
    )
    def k(x_hbm, flags_hbm, out_hbm, flags_v, *bufs_sems):
        bufs = bufs_sems[:D]
        isems = bufs_sems[D:2 * D]
        osems = bufs_sems[2 * D:]
        wid = lax.axis_index("s") * NC + lax.axis_index("c")
        gbase = wid * per_w
        pltpu.sync_copy(flags_hbm.at[pl.ds(gbase, per_w)],
                        flags_v.at[pl.ds(0, per_w)])

        def flag_of(u):
            return flags_v[pl.ds(u // NS_IMG, 16)][0]

        def in_descr(u, p):
            g = gbase + u // NS_IMG
            return pltpu.make_async_copy(
                x_hbm.at[g, pl.ds(Q * (u % NS_IMG), Q)], bufs[p], isems[p]
            )

        def out_descr(u, p, flag):
            g = gbase + u // NS_IMG
            q = u % NS_IMG
            off = jnp.where(flag != 0, Q * (NS_IMG - 1 - q), Q * q)
            return pltpu.make_async_copy(
                bufs[p], out_hbm.at[g, pl.ds(off, Q)], osems[p]
            )

        def reverse(buf):
            def swap(j, carry):
                for kk in range(W // 16):
                    sl = pl.ds(16 * kk, 16)
                    a = buf[j, sl]
                    b = buf[Q - 1 - j, sl]
                    buf[j, sl] = b
                    buf[Q - 1 - j, sl] = a
                return carry
            lax.fori_loop(0, Q // 2, swap, 0)

        for p0 in range(4):
            in_descr(p0, p0).start()

        def body(i, carry):
            for p in range(D):
                u = D * i + p
                flag = flag_of(u)
                in_descr(u, p).wait()

                @pl.when(flag != 0)
                def _rev():
                    reverse(bufs[p])

                out_descr(u, p, flag).start()

                @pl.when(u + 4 < n_units)
                def _prefetch():
                    pn = (p + 4) % D

                    @pl.when(u >= 4)
                    def _drain():
                        out_descr(u - 4, pn, flag_of(u - 4)).wait()

                    in_descr(u + 4, pn).start()

            return carry

        lax.fori_loop(0, n_units // D, body, 0)
        for t in range(4):
            u = n_units - 4 + t
            out_descr(u, u % D, flag_of(u)).wait()

    return k(x_imgs, flags)


def _kernel_sc(x, params, indices):
    B, C, H, W = x.shape
    mask = jnp.zeros((C,), jnp.bool_).at[indices].set(True)
    mask = jnp.logical_and(mask, params[0])
    flags = jnp.tile(mask.astype(jnp.int32), B)
    out = _sc_flip_images(x.reshape(B * C, H, W), flags)
    return out.reshape(B, C, H, W)


def kernel(x, params, indices):
    return _kernel_sc(x, params, indices)
